# baseline (speedup 1.0000x reference)
import functools

import jax
import jax.numpy as jnp
from jax import lax
from jax.experimental import pallas as pl
from jax.experimental.pallas import tpu as pltpu
from jax.experimental.pallas import tpu_sc as plsc

NC = 2
NS = 16
LANES = 16


def _dense_support(features, W_emb, b_emb, W_gc1):
    n, f = features.shape
    d = W_emb.shape[1]
    half = d // 2
    rb = 1000

    def body(x_ref, we_ref, be_ref, wg_ref, out_ref):
        x = x_ref[...]
        h = jnp.maximum(
            jnp.dot(x, we_ref[...], preferred_element_type=jnp.float32)
            + be_ref[...], 0.0)
        s = jnp.maximum(
            jnp.dot(h, wg_ref[...], preferred_element_type=jnp.float32), 0.0)
        out_ref[0] = s[:, :half]
        out_ref[1] = s[:, half:]

    return pl.pallas_call(
        body,
        grid=(n // rb,),
        in_specs=[
            pl.BlockSpec((rb, f), lambda i: (i, 0)),
            pl.BlockSpec((f, d), lambda i: (0, 0)),
            pl.BlockSpec((1, d), lambda i: (0, 0)),
            pl.BlockSpec((d, d), lambda i: (0, 0)),
        ],
        out_specs=pl.BlockSpec((2, rb, half), lambda i: (0, i, 0)),
        out_shape=jax.ShapeDtypeStruct((2, n, half), jnp.float32),
    )(features, W_emb, b_emb.reshape(1, d), W_gc1)


def _spmm_sc(rows3, cols4, w16, x_flat, ztile, n):
    ns_, nch, ck = rows3.shape
    sup = 32
    nsup = nch // sup
    ept = nch * ck
    rpt = (n // NS) // 8 * 8
    tail = n - NS * rpt
    half = x_flat.shape[1]
    mesh = plsc.VectorSubcoreMesh(core_axis_name="c", subcore_axis_name="s")

    @functools.partial(
        pl.kernel,
        mesh=mesh,
        out_type=jax.ShapeDtypeStruct((2 * n, half), jnp.float32),
        scratch_types=[
            pltpu.VMEM((sup, ck), jnp.int32),
            pltpu.VMEM((sup, ck), jnp.int32),
            pltpu.VMEM((ck, LANES), jnp.float32),
            pltpu.VMEM((ck, LANES), jnp.float32),
            pltpu.VMEM((ck, LANES), jnp.float32),
            pltpu.VMEM((ck, half), jnp.float32),
            pltpu.VMEM((ck, half), jnp.float32),
            pltpu.VMEM((ck, half), jnp.float32),
            pltpu.VMEM_SHARED((n, half), jnp.float32),
        ] + [pltpu.SemaphoreType.DMA] * 9,
    )
    def k(rows_hbm, cols_hbm, w_hbm, x_hbm, z_hbm, out_hbm,
          rows_v, cols_v, wbuf0, wbuf1, wbuf2, buf0, buf1, buf2,
          acc_sh, gsem0, gsem1, gsem2, wsem0, wsem1, wsem2,
          ssem0, ssem1, ssem2):
        c = lax.axis_index("c")
        s = lax.axis_index("s")
        pltpu.sync_copy(z_hbm, acc_sh.at[pl.ds(s * rpt, rpt)])
        if tail:
            @pl.when(s == 0)
            def _init_tail():
                pltpu.sync_copy(z_hbm.at[:tail],
                                acc_sh.at[pl.ds(NS * rpt, tail)])
        plsc.subcore_barrier()

        bufs = (buf0, buf1, buf2)
        wbufs = (wbuf0, wbuf1, wbuf2)
        gsems = (gsem0, gsem1, gsem2)
        wsems = (wsem0, wsem1, wsem2)
        ssems = (ssem0, ssem1, ssem2)

        def wslice(gci):
            return w_hbm.at[pl.ds(s * ept + gci * ck, ck)]

        def gstart(r, ci, base):
            pltpu.async_copy(x_hbm.at[cols_v.at[ci]], bufs[r], gsems[r])
            pltpu.async_copy(wslice(base + ci), wbufs[r], wsems[r])

        def gwait(r, ci, base):
            pltpu.make_async_copy(x_hbm.at[cols_v.at[ci]], bufs[r],
                                  gsems[r]).wait()
            pltpu.make_async_copy(wslice(base + ci), wbufs[r], wsems[r]).wait()

        def swait(r, ci):
            pltpu.make_async_copy(bufs[r], acc_sh.at[rows_v.at[ci]],
                                  ssems[r]).wait()

        def scale(buf, wbuf):
            def edge_body(i, inner):
                wspl = wbuf[i]
                for j in range(half // LANES):
                    sl = pl.ds(j * LANES, LANES)
                    buf[i, sl] = buf[i, sl] * wspl
                return inner

            lax.fori_loop(0, ck, edge_body, 0, unroll=4)

        def _cond(pred, fn):
            if isinstance(pred, bool):
                if pred:
                    fn()
            else:
                pl.when(pred)(fn)

        def chunk_ops(ci, r, base):
            gwait(r, ci, base)
            scale(bufs[r], wbufs[r])
            pltpu.async_copy(bufs[r], acc_sh.at[rows_v.at[ci]], ssems[r],
                             add=True)
            r2 = (r + 2) % 3
            _cond(ci >= 1, lambda: swait(r2, ci - 1))
            _cond(ci + 2 < sup, lambda: gstart(r2, ci + 2, base))

        def super_body(si, carry):
            pltpu.sync_copy(rows_hbm.at[s, pl.ds(si * sup, sup)], rows_v)
            pltpu.sync_copy(cols_hbm.at[c, s, pl.ds(si * sup, sup)], cols_v)
            base = si * sup
            gstart(0, 0, base)
            gstart(1, 1, base)

            def triple_body(t, carry2):
                for dr in range(3):
                    chunk_ops(3 * t + dr, dr, base)
                return carry2

            lax.fori_loop(0, sup // 3, triple_body, 0)
            for ci in range(sup // 3 * 3, sup):
                chunk_ops(ci, ci % 3, base)
            swait((sup - 1) % 3, sup - 1)
            return carry

        lax.fori_loop(0, nsup, super_body, 0)
        plsc.subcore_barrier()
        pltpu.sync_copy(acc_sh.at[pl.ds(s * rpt, rpt)],
                        out_hbm.at[pl.ds(c * n + s * rpt, rpt)])
        if tail:
            @pl.when(s == 0)
            def _out_tail():
                pltpu.sync_copy(acc_sh.at[pl.ds(NS * rpt, tail)],
                                out_hbm.at[pl.ds(c * n + NS * rpt, tail)])

    return k(rows3, cols4, w16, x_flat, ztile)


def _mix(support3, as3, aas3, b2):
    _, n, half = support3.shape
    rb = 1000

    def body(s_ref, a_ref, aa_ref, b_ref, out_ref):
        S = s_ref[...]
        A = a_ref[...]
        AA = aa_ref[...]
        low = A + S
        mid = AA - S
        high = S - A
        o1 = high * jnp.maximum(low + mid, 0.0)
        o2 = mid * jnp.maximum(low + high, 0.0)
        o3 = low * jnp.maximum(high + mid, 0.0)
        out_ref[...] = jnp.maximum(o1 + o2 + o3, 0.0) + b_ref[...]

    return pl.pallas_call(
        body,
        grid=(n // rb,),
        in_specs=[pl.BlockSpec((2, rb, half), lambda i: (0, i, 0))] * 3
        + [pl.BlockSpec((2, 1, half), lambda i: (0, 0, 0))],
        out_specs=pl.BlockSpec((2, rb, half), lambda i: (0, i, 0)),
        out_shape=jax.ShapeDtypeStruct((2, n, half), jnp.float32),
    )(support3, as3, aas3, b2)


def _bpr_sc(lat_flat, idx6, b):
    half = lat_flat.shape[1]
    nw = NC * NS
    bpw = b // nw
    sub = 64
    nsub = bpw // sub
    mesh = plsc.VectorSubcoreMesh(core_axis_name="c", subcore_axis_name="s")

    @functools.partial(
        pl.kernel,
        mesh=mesh,
        out_type=[jax.ShapeDtypeStruct((b, half), jnp.float32),
                  jax.ShapeDtypeStruct((b, half), jnp.float32)],
        scratch_types=[pltpu.VMEM((6, bpw), jnp.int32)]
        + [pltpu.VMEM((sub, half), jnp.float32) for _ in range(8)]
        + [pltpu.SemaphoreType.DMA],
    )
    def k(lat_hbm, idx_hbm, psp_hbm, nsp_hbm,
          idx_v, k0, p0, n0, k1, p1, n1, pbuf, nbuf, sem):
        cc = lax.axis_index("c")
        ss = lax.axis_index("s")
        base = (ss * NC + cc) * bpw
        bufs = (k0, p0, n0, k1, p1, n1)
        pltpu.sync_copy(idx_hbm.at[:, pl.ds(base, bpw)], idx_v)

        def sub_body(si, carry):
            for j in range(6):
                pltpu.async_copy(
                    lat_hbm.at[idx_v.at[j, pl.ds(si * sub, sub)]],
                    bufs[j], sem).wait()

            def row_body(r, inner):
                for j in range(half // LANES):
                    sl = pl.ds(j * LANES, LANES)
                    kv0 = k0[r, sl]
                    kv1 = k1[r, sl]
                    pbuf[r, sl] = kv0 * p0[r, sl] + kv1 * p1[r, sl]
                    nbuf[r, sl] = kv0 * n0[r, sl] + kv1 * n1[r, sl]
                return inner

            lax.fori_loop(0, sub, row_body, 0, unroll=2)
            pltpu.sync_copy(pbuf, psp_hbm.at[pl.ds(base + si * sub, sub)])
            pltpu.sync_copy(nbuf, nsp_hbm.at[pl.ds(base + si * sub, sub)])
            return carry

        lax.fori_loop(0, nsub, sub_body, 0)

    return k(lat_flat, idx6)


def _metrics(psp3, nsp3):
    m, r, c = psp3.shape

    def body(ps_ref, ns_ref, out_ref):
        ps = jnp.sum(ps_ref[...], axis=2)
        ns = jnp.sum(ns_ref[...], axis=2)
        d = ps - ns
        e = jnp.exp(-jnp.abs(d))
        sig = jnp.where(d >= 0, 1.0 / (1.0 + e), e / (1.0 + e))
        lterm = jnp.log(sig + 1e-9)
        nb = jnp.float32(m * r)
        loss = -jnp.sum(lterm) / nb
        win = ps >= ns
        hr = jnp.sum(win.astype(jnp.float32)) / nb
        mrr = jnp.sum(jnp.where(win, jnp.float32(1e-9),
                                jnp.float32(1.0))) / nb
        ndcg = jnp.sum(jnp.where(win, jnp.float32(1.0),
                                 jnp.float32(2.0 / 3.0))) / nb
        row = lax.broadcasted_iota(jnp.int32, (8, 128), 0)
        lane = lax.broadcasted_iota(jnp.int32, (8, 128), 1)
        z = jnp.zeros((8, 128), jnp.float32)
        v = jnp.where((row == 0) & (lane == 0), loss, z)
        v = jnp.where((row == 0) & (lane == 1), mrr, v)
        v = jnp.where((row == 0) & (lane == 2), hr, v)
        v = jnp.where((row == 0) & (lane == 3), ndcg, v)
        out_ref[...] = v

    return pl.pallas_call(
        body,
        in_specs=[pl.BlockSpec((m, r, c), lambda: (0, 0, 0))] * 2,
        out_specs=pl.BlockSpec((8, 128), lambda: (0, 0)),
        out_shape=jax.ShapeDtypeStruct((8, 128), jnp.float32),
    )(psp3, nsp3)


def kernel(features, edge_index, edge_weight, train_set, W_emb, b_emb,
           W_gc1, b_gc1):
    n, _ = features.shape
    d = W_emb.shape[1]
    half = d // 2
    e = edge_index.shape[1]
    b = train_set.shape[0]

    support3 = _dense_support(features, W_emb, b_emb, W_gc1)
    sflat = support3.reshape(2 * n, half)

    ck = 40
    blk = ck * 32 * NS
    e2 = (e + blk - 1) // blk * blk
    pad = e2 - e
    rows = jnp.concatenate([edge_index[0], jnp.zeros((pad,), jnp.int32)])
    cols = jnp.concatenate([edge_index[1], jnp.zeros((pad,), jnp.int32)])
    wpad = jnp.concatenate([edge_weight, jnp.zeros((pad,), jnp.float32)])
    ept = e2 // NS
    nch = ept // ck
    rows3 = rows.reshape(NS, nch, ck)
    cols4 = jnp.stack([cols, cols + n]).reshape(NC, NS, nch, ck)
    ztile = jnp.zeros(((n // NS) // 8 * 8, half), jnp.float32)

    w16 = jnp.broadcast_to(wpad[:, None], (e2, LANES))
    as_flat = _spmm_sc(rows3, cols4, w16, sflat, ztile, n)
    aas_flat = _spmm_sc(rows3, cols4, w16, as_flat, ztile, n)

    latent3 = _mix(support3, as_flat.reshape(2, n, half),
                   aas_flat.reshape(2, n, half), b_gc1.reshape(2, 1, half))

    kk = train_set[:, 0]
    pp = train_set[:, 1]
    nn = train_set[:, 2]
    idx6 = jnp.stack([kk, pp, nn, kk + n, pp + n, nn + n])
    psp, nsp = _bpr_sc(latent3.reshape(2 * n, half), idx6, b)

    out8 = _metrics(psp.reshape(b // 128, 128, half),
                    nsp.reshape(b // 128, 128, half))
    return (out8[0, 0], out8[0, 1], out8[0, 2], out8[0, 3])


# speedup vs baseline: 1.1205x; 1.1205x over previous
import functools

import jax
import jax.numpy as jnp
from jax import lax
from jax.experimental import pallas as pl
from jax.experimental.pallas import tpu as pltpu
from jax.experimental.pallas import tpu_sc as plsc

NC = 2
NS = 16
LANES = 16


def _dense_support(features, W_emb, b_emb, W_gc1):
    n, f = features.shape
    d = W_emb.shape[1]
    half = d // 2
    rb = 1000

    def body(x_ref, we_ref, be_ref, wg_ref, out_ref):
        x = x_ref[...]
        h = jnp.maximum(
            jnp.dot(x, we_ref[...], preferred_element_type=jnp.float32)
            + be_ref[...], 0.0)
        s = jnp.maximum(
            jnp.dot(h, wg_ref[...], preferred_element_type=jnp.float32), 0.0)
        out_ref[0] = s[:, :half]
        out_ref[1] = s[:, half:]

    return pl.pallas_call(
        body,
        grid=(n // rb,),
        in_specs=[
            pl.BlockSpec((rb, f), lambda i: (i, 0)),
            pl.BlockSpec((f, d), lambda i: (0, 0)),
            pl.BlockSpec((1, d), lambda i: (0, 0)),
            pl.BlockSpec((d, d), lambda i: (0, 0)),
        ],
        out_specs=pl.BlockSpec((2, rb, half), lambda i: (0, i, 0)),
        out_shape=jax.ShapeDtypeStruct((2, n, half), jnp.float32),
    )(features, W_emb, b_emb.reshape(1, d), W_gc1)


def _spmm_sc(rows3, cols4, w16, x_flat, ztile, n):
    ns_, nch, ck = rows3.shape
    sup = 32
    nsup = nch // sup
    ept = nch * ck
    rpt = (n // NS) // 8 * 8
    tail = n - NS * rpt
    half = x_flat.shape[1]
    mesh = plsc.VectorSubcoreMesh(core_axis_name="c", subcore_axis_name="s")

    @functools.partial(
        pl.kernel,
        mesh=mesh,
        out_type=jax.ShapeDtypeStruct((2 * n, half), jnp.float32),
        scratch_types=[
            pltpu.VMEM((sup, ck), jnp.int32),
            pltpu.VMEM((sup, ck), jnp.int32),
            pltpu.VMEM((ck, LANES), jnp.float32),
            pltpu.VMEM((ck, LANES), jnp.float32),
            pltpu.VMEM((ck, half), jnp.float32),
            pltpu.VMEM((ck, half), jnp.float32),
            pltpu.VMEM_SHARED((n, half), jnp.float32),
            pltpu.SemaphoreType.DMA,
            pltpu.SemaphoreType.DMA,
            pltpu.SemaphoreType.DMA,
            pltpu.SemaphoreType.DMA,
        ],
    )
    def k(rows_hbm, cols_hbm, w_hbm, x_hbm, z_hbm, out_hbm,
          rows_v, cols_v, wbuf0, wbuf1, buf0, buf1,
          acc_sh, gsem0, gsem1, wsem0, wsem1):
        c = lax.axis_index("c")
        s = lax.axis_index("s")
        pltpu.sync_copy(z_hbm, acc_sh.at[pl.ds(s * rpt, rpt)])
        if tail:
            @pl.when(s == 0)
            def _init_tail():
                pltpu.sync_copy(z_hbm.at[:tail],
                                acc_sh.at[pl.ds(NS * rpt, tail)])
        plsc.subcore_barrier()

        def wslice(gci):
            return w_hbm.at[pl.ds(s * ept + gci * ck, ck)]

        def scale_scatter(buf, wbuf, ci):
            def edge_body(i, inner):
                wspl = wbuf[i]
                for j in range(half // LANES):
                    sl = pl.ds(j * LANES, LANES)
                    buf[i, sl] = buf[i, sl] * wspl
                return inner

            lax.fori_loop(0, ck, edge_body, 0, unroll=4)
            pltpu.sync_copy(buf, acc_sh.at[rows_v.at[ci]], add=True)

        def super_body(si, carry):
            pltpu.sync_copy(rows_hbm.at[s, pl.ds(si * sup, sup)], rows_v)
            pltpu.sync_copy(cols_hbm.at[c, s, pl.ds(si * sup, sup)], cols_v)
            base = si * sup
            pltpu.async_copy(x_hbm.at[cols_v.at[0]], buf0, gsem0)
            pltpu.async_copy(wslice(base), wbuf0, wsem0)

            def pair_body(cp, carry2):
                a = 2 * cp
                b = a + 1
                pltpu.make_async_copy(x_hbm.at[cols_v.at[a]], buf0,
                                      gsem0).wait()
                pltpu.make_async_copy(wslice(base + a), wbuf0, wsem0).wait()
                pltpu.async_copy(x_hbm.at[cols_v.at[b]], buf1, gsem1)
                pltpu.async_copy(wslice(base + b), wbuf1, wsem1)
                scale_scatter(buf0, wbuf0, a)
                pltpu.make_async_copy(x_hbm.at[cols_v.at[b]], buf1,
                                      gsem1).wait()
                pltpu.make_async_copy(wslice(base + b), wbuf1, wsem1).wait()

                @pl.when(cp + 1 < sup // 2)
                def _prefetch_next():
                    pltpu.async_copy(x_hbm.at[cols_v.at[a + 2]], buf0, gsem0)
                    pltpu.async_copy(wslice(base + a + 2), wbuf0, wsem0)

                scale_scatter(buf1, wbuf1, b)
                return carry2

            lax.fori_loop(0, sup // 2, pair_body, 0)
            return carry

        lax.fori_loop(0, nsup, super_body, 0)
        plsc.subcore_barrier()
        pltpu.sync_copy(acc_sh.at[pl.ds(s * rpt, rpt)],
                        out_hbm.at[pl.ds(c * n + s * rpt, rpt)])
        if tail:
            @pl.when(s == 0)
            def _out_tail():
                pltpu.sync_copy(acc_sh.at[pl.ds(NS * rpt, tail)],
                                out_hbm.at[pl.ds(c * n + NS * rpt, tail)])

    return k(rows3, cols4, w16, x_flat, ztile)


def _mix(support3, as3, aas3, b2):
    _, n, half = support3.shape
    rb = 1000

    def body(s_ref, a_ref, aa_ref, b_ref, out_ref):
        S = s_ref[...]
        A = a_ref[...]
        AA = aa_ref[...]
        low = A + S
        mid = AA - S
        high = S - A
        o1 = high * jnp.maximum(low + mid, 0.0)
        o2 = mid * jnp.maximum(low + high, 0.0)
        o3 = low * jnp.maximum(high + mid, 0.0)
        out_ref[...] = jnp.maximum(o1 + o2 + o3, 0.0) + b_ref[...]

    return pl.pallas_call(
        body,
        grid=(n // rb,),
        in_specs=[pl.BlockSpec((2, rb, half), lambda i: (0, i, 0))] * 3
        + [pl.BlockSpec((2, 1, half), lambda i: (0, 0, 0))],
        out_specs=pl.BlockSpec((2, rb, half), lambda i: (0, i, 0)),
        out_shape=jax.ShapeDtypeStruct((2, n, half), jnp.float32),
    )(support3, as3, aas3, b2)


def _bpr_sc(lat_flat, idx6, b):
    half = lat_flat.shape[1]
    nw = NC * NS
    bpw = b // nw
    sub = 32
    nsub = bpw // sub
    mesh = plsc.VectorSubcoreMesh(core_axis_name="c", subcore_axis_name="s")

    @functools.partial(
        pl.kernel,
        mesh=mesh,
        out_type=[jax.ShapeDtypeStruct((b, half), jnp.float32),
                  jax.ShapeDtypeStruct((b, half), jnp.float32)],
        scratch_types=[pltpu.VMEM((6, bpw), jnp.int32),
                       pltpu.VMEM((2, 6, sub, half), jnp.float32),
                       pltpu.VMEM((2, 2, sub, half), jnp.float32)]
        + [pltpu.SemaphoreType.DMA] * 6,
    )
    def k(lat_hbm, idx_hbm, psp_hbm, nsp_hbm,
          idx_v, gb, ob, gsem0, gsem1, psem0, psem1, nsem0, nsem1):
        cc = lax.axis_index("c")
        ss = lax.axis_index("s")
        base = (ss * NC + cc) * bpw
        gsems = (gsem0, gsem1)
        psems = (psem0, psem1)
        nsems = (nsem0, nsem1)
        pltpu.sync_copy(idx_hbm.at[:, pl.ds(base, bpw)], idx_v)

        def gfire(h, si):
            for j in range(6):
                pltpu.async_copy(
                    lat_hbm.at[idx_v.at[j, pl.ds(si * sub, sub)]],
                    gb.at[h, j], gsems[h])

        def gdrain(h, si):
            for j in range(6):
                pltpu.make_async_copy(
                    lat_hbm.at[idx_v.at[j, pl.ds(si * sub, sub)]],
                    gb.at[h, j], gsems[h]).wait()

        def compute(h):
            k0, p0, n0, k1, p1, n1 = (gb.at[h, j] for j in range(6))
            pbuf = ob.at[h, 0]
            nbuf = ob.at[h, 1]

            def row_body(r, inner):
                for j in range(half // LANES):
                    sl = pl.ds(j * LANES, LANES)
                    kv0 = k0[r, sl]
                    kv1 = k1[r, sl]
                    pbuf[r, sl] = kv0 * p0[r, sl] + kv1 * p1[r, sl]
                    nbuf[r, sl] = kv0 * n0[r, sl] + kv1 * n1[r, sl]
                return inner

            lax.fori_loop(0, sub, row_body, 0, unroll=4)

        def owait(h, si):
            pltpu.make_async_copy(
                ob.at[h, 0], psp_hbm.at[pl.ds(base + si * sub, sub)],
                psems[h]).wait()
            pltpu.make_async_copy(
                ob.at[h, 1], nsp_hbm.at[pl.ds(base + si * sub, sub)],
                nsems[h]).wait()

        def ofire(h, si):
            pltpu.async_copy(ob.at[h, 0],
                             psp_hbm.at[pl.ds(base + si * sub, sub)],
                             psems[h])
            pltpu.async_copy(ob.at[h, 1],
                             nsp_hbm.at[pl.ds(base + si * sub, sub)],
                             nsems[h])

        gfire(0, 0)

        def pair_body(q, carry):
            a = 2 * q
            gdrain(0, a)
            gfire(1, a + 1)

            @pl.when(q >= 1)
            def _w0():
                owait(0, a - 2)

            compute(0)
            ofire(0, a)
            gdrain(1, a + 1)

            @pl.when(q + 1 < nsub // 2)
            def _g0():
                gfire(0, a + 2)

            @pl.when(q >= 1)
            def _w1():
                owait(1, a - 1)

            compute(1)
            ofire(1, a + 1)
            return carry

        lax.fori_loop(0, nsub // 2, pair_body, 0)
        owait(0, nsub - 2)
        owait(1, nsub - 1)

    return k(lat_flat, idx6)


def _metrics(psp3, nsp3):
    m, r, c = psp3.shape

    def body(ps_ref, ns_ref, out_ref):
        ps = jnp.sum(ps_ref[...], axis=2)
        ns = jnp.sum(ns_ref[...], axis=2)
        d = ps - ns
        e = jnp.exp(-jnp.abs(d))
        sig = jnp.where(d >= 0, 1.0 / (1.0 + e), e / (1.0 + e))
        lterm = jnp.log(sig + 1e-9)
        nb = jnp.float32(m * r)
        loss = -jnp.sum(lterm) / nb
        win = ps >= ns
        hr = jnp.sum(win.astype(jnp.float32)) / nb
        mrr = jnp.sum(jnp.where(win, jnp.float32(1e-9),
                                jnp.float32(1.0))) / nb
        ndcg = jnp.sum(jnp.where(win, jnp.float32(1.0),
                                 jnp.float32(2.0 / 3.0))) / nb
        row = lax.broadcasted_iota(jnp.int32, (8, 128), 0)
        lane = lax.broadcasted_iota(jnp.int32, (8, 128), 1)
        z = jnp.zeros((8, 128), jnp.float32)
        v = jnp.where((row == 0) & (lane == 0), loss, z)
        v = jnp.where((row == 0) & (lane == 1), mrr, v)
        v = jnp.where((row == 0) & (lane == 2), hr, v)
        v = jnp.where((row == 0) & (lane == 3), ndcg, v)
        out_ref[...] = v

    return pl.pallas_call(
        body,
        in_specs=[pl.BlockSpec((m, r, c), lambda: (0, 0, 0))] * 2,
        out_specs=pl.BlockSpec((8, 128), lambda: (0, 0)),
        out_shape=jax.ShapeDtypeStruct((8, 128), jnp.float32),
    )(psp3, nsp3)


def kernel(features, edge_index, edge_weight, train_set, W_emb, b_emb,
           W_gc1, b_gc1):
    n, _ = features.shape
    d = W_emb.shape[1]
    half = d // 2
    e = edge_index.shape[1]
    b = train_set.shape[0]

    support3 = _dense_support(features, W_emb, b_emb, W_gc1)
    sflat = support3.reshape(2 * n, half)

    ck = 80
    blk = ck * 32 * NS
    e2 = (e + blk - 1) // blk * blk
    pad = e2 - e
    rows = jnp.concatenate([edge_index[0], jnp.zeros((pad,), jnp.int32)])
    cols = jnp.concatenate([edge_index[1], jnp.zeros((pad,), jnp.int32)])
    wpad = jnp.concatenate([edge_weight, jnp.zeros((pad,), jnp.float32)])
    ept = e2 // NS
    nch = ept // ck
    rows3 = rows.reshape(NS, nch, ck)
    cols4 = jnp.stack([cols, cols + n]).reshape(NC, NS, nch, ck)
    ztile = jnp.zeros(((n // NS) // 8 * 8, half), jnp.float32)

    w16 = jnp.broadcast_to(wpad[:, None], (e2, LANES))
    as_flat = _spmm_sc(rows3, cols4, w16, sflat, ztile, n)
    aas_flat = _spmm_sc(rows3, cols4, w16, as_flat, ztile, n)

    latent3 = _mix(support3, as_flat.reshape(2, n, half),
                   aas_flat.reshape(2, n, half), b_gc1.reshape(2, 1, half))

    kk = train_set[:, 0]
    pp = train_set[:, 1]
    nn = train_set[:, 2]
    idx6 = jnp.stack([kk, pp, nn, kk + n, pp + n, nn + n])
    psp, nsp = _bpr_sc(latent3.reshape(2 * n, half), idx6, b)

    out8 = _metrics(psp.reshape(b // 128, 128, half),
                    nsp.reshape(b // 128, 128, half))
    return (out8[0, 0], out8[0, 1], out8[0, 2], out8[0, 3])


# speedup vs baseline: 1.1211x; 1.0005x over previous
import functools

import jax
import jax.numpy as jnp
from jax import lax
from jax.experimental import pallas as pl
from jax.experimental.pallas import tpu as pltpu
from jax.experimental.pallas import tpu_sc as plsc

NC = 2
NS = 16
LANES = 16


def _dense_support(features, W_emb, b_emb, W_gc1):
    n, f = features.shape
    d = W_emb.shape[1]
    half = d // 2
    rb = 1000

    def body(x_ref, we_ref, be_ref, wg_ref, out_ref):
        x = x_ref[...]
        h = jnp.maximum(
            jnp.dot(x, we_ref[...], preferred_element_type=jnp.float32)
            + be_ref[...], 0.0)
        s = jnp.maximum(
            jnp.dot(h, wg_ref[...], preferred_element_type=jnp.float32), 0.0)
        out_ref[0] = s[:, :half]
        out_ref[1] = s[:, half:]

    return pl.pallas_call(
        body,
        grid=(n // rb,),
        in_specs=[
            pl.BlockSpec((rb, f), lambda i: (i, 0)),
            pl.BlockSpec((f, d), lambda i: (0, 0)),
            pl.BlockSpec((1, d), lambda i: (0, 0)),
            pl.BlockSpec((d, d), lambda i: (0, 0)),
        ],
        out_specs=pl.BlockSpec((2, rb, half), lambda i: (0, i, 0)),
        out_shape=jax.ShapeDtypeStruct((2, n, half), jnp.float32),
    )(features, W_emb, b_emb.reshape(1, d), W_gc1)


def _spmm_sc(rows3, cols4, w16, x_flat, ztile, n):
    ns_, nch, ck = rows3.shape
    sup = 32
    nsup = nch // sup
    ept = nch * ck
    rpt = (n // NS) // 8 * 8
    tail = n - NS * rpt
    half = x_flat.shape[1]
    mesh = plsc.VectorSubcoreMesh(core_axis_name="c", subcore_axis_name="s")

    @functools.partial(
        pl.kernel,
        mesh=mesh,
        out_type=jax.ShapeDtypeStruct((2 * n, half), jnp.float32),
        scratch_types=[
            pltpu.VMEM((sup, ck), jnp.int32),
            pltpu.VMEM((sup, ck), jnp.int32),
            pltpu.VMEM((ck, LANES), jnp.float32),
            pltpu.VMEM((ck, LANES), jnp.float32),
            pltpu.VMEM((ck, half), jnp.float32),
            pltpu.VMEM((ck, half), jnp.float32),
            pltpu.VMEM_SHARED((n, half), jnp.float32),
            pltpu.SemaphoreType.DMA,
            pltpu.SemaphoreType.DMA,
            pltpu.SemaphoreType.DMA,
            pltpu.SemaphoreType.DMA,
        ],
    )
    def k(rows_hbm, cols_hbm, w_hbm, x_hbm, z_hbm, out_hbm,
          rows_v, cols_v, wbuf0, wbuf1, buf0, buf1,
          acc_sh, gsem0, gsem1, wsem0, wsem1):
        c = lax.axis_index("c")
        s = lax.axis_index("s")
        pltpu.sync_copy(z_hbm, acc_sh.at[pl.ds(s * rpt, rpt)])
        if tail:
            @pl.when(s == 0)
            def _init_tail():
                pltpu.sync_copy(z_hbm.at[:tail],
                                acc_sh.at[pl.ds(NS * rpt, tail)])
        plsc.subcore_barrier()

        def wslice(gci):
            return w_hbm.at[pl.ds(s * ept + gci * ck, ck)]

        def scale_scatter(buf, wbuf, ci):
            def edge_body(i, inner):
                wspl = wbuf[i]
                for j in range(half // LANES):
                    sl = pl.ds(j * LANES, LANES)
                    buf[i, sl] = buf[i, sl] * wspl
                return inner

            lax.fori_loop(0, ck, edge_body, 0, unroll=8)
            pltpu.sync_copy(buf, acc_sh.at[rows_v.at[ci]], add=True)

        def super_body(si, carry):
            pltpu.sync_copy(rows_hbm.at[s, pl.ds(si * sup, sup)], rows_v)
            pltpu.sync_copy(cols_hbm.at[c, s, pl.ds(si * sup, sup)], cols_v)
            base = si * sup
            pltpu.async_copy(x_hbm.at[cols_v.at[0]], buf0, gsem0)
            pltpu.async_copy(wslice(base), wbuf0, wsem0)

            def pair_body(cp, carry2):
                a = 2 * cp
                b = a + 1
                pltpu.make_async_copy(x_hbm.at[cols_v.at[a]], buf0,
                                      gsem0).wait()
                pltpu.make_async_copy(wslice(base + a), wbuf0, wsem0).wait()
                pltpu.async_copy(x_hbm.at[cols_v.at[b]], buf1, gsem1)
                pltpu.async_copy(wslice(base + b), wbuf1, wsem1)
                scale_scatter(buf0, wbuf0, a)
                pltpu.make_async_copy(x_hbm.at[cols_v.at[b]], buf1,
                                      gsem1).wait()
                pltpu.make_async_copy(wslice(base + b), wbuf1, wsem1).wait()

                @pl.when(cp + 1 < sup // 2)
                def _prefetch_next():
                    pltpu.async_copy(x_hbm.at[cols_v.at[a + 2]], buf0, gsem0)
                    pltpu.async_copy(wslice(base + a + 2), wbuf0, wsem0)

                scale_scatter(buf1, wbuf1, b)
                return carry2

            lax.fori_loop(0, sup // 2, pair_body, 0)
            return carry

        lax.fori_loop(0, nsup, super_body, 0)
        plsc.subcore_barrier()
        pltpu.sync_copy(acc_sh.at[pl.ds(s * rpt, rpt)],
                        out_hbm.at[pl.ds(c * n + s * rpt, rpt)])
        if tail:
            @pl.when(s == 0)
            def _out_tail():
                pltpu.sync_copy(acc_sh.at[pl.ds(NS * rpt, tail)],
                                out_hbm.at[pl.ds(c * n + NS * rpt, tail)])

    return k(rows3, cols4, w16, x_flat, ztile)


def _mix(support3, as3, aas3, b2):
    _, n, half = support3.shape
    rb = 1000

    def body(s_ref, a_ref, aa_ref, b_ref, out_ref):
        S = s_ref[...]
        A = a_ref[...]
        AA = aa_ref[...]
        low = A + S
        mid = AA - S
        high = S - A
        o1 = high * jnp.maximum(low + mid, 0.0)
        o2 = mid * jnp.maximum(low + high, 0.0)
        o3 = low * jnp.maximum(high + mid, 0.0)
        out_ref[...] = jnp.maximum(o1 + o2 + o3, 0.0) + b_ref[...]

    return pl.pallas_call(
        body,
        grid=(n // rb,),
        in_specs=[pl.BlockSpec((2, rb, half), lambda i: (0, i, 0))] * 3
        + [pl.BlockSpec((2, 1, half), lambda i: (0, 0, 0))],
        out_specs=pl.BlockSpec((2, rb, half), lambda i: (0, i, 0)),
        out_shape=jax.ShapeDtypeStruct((2, n, half), jnp.float32),
    )(support3, as3, aas3, b2)


def _bpr_sc(lat_flat, idx6, b):
    half = lat_flat.shape[1]
    nw = NC * NS
    bpw = b // nw
    sub = 32
    nsub = bpw // sub
    mesh = plsc.VectorSubcoreMesh(core_axis_name="c", subcore_axis_name="s")

    @functools.partial(
        pl.kernel,
        mesh=mesh,
        out_type=[jax.ShapeDtypeStruct((b, half), jnp.float32),
                  jax.ShapeDtypeStruct((b, half), jnp.float32)],
        scratch_types=[pltpu.VMEM((6, bpw), jnp.int32),
                       pltpu.VMEM((2, 6, sub, half), jnp.float32),
                       pltpu.VMEM((2, 2, sub, half), jnp.float32)]
        + [pltpu.SemaphoreType.DMA] * 6,
    )
    def k(lat_hbm, idx_hbm, psp_hbm, nsp_hbm,
          idx_v, gb, ob, gsem0, gsem1, psem0, psem1, nsem0, nsem1):
        cc = lax.axis_index("c")
        ss = lax.axis_index("s")
        base = (ss * NC + cc) * bpw
        gsems = (gsem0, gsem1)
        psems = (psem0, psem1)
        nsems = (nsem0, nsem1)
        pltpu.sync_copy(idx_hbm.at[:, pl.ds(base, bpw)], idx_v)

        def gfire(h, si):
            for j in range(6):
                pltpu.async_copy(
                    lat_hbm.at[idx_v.at[j, pl.ds(si * sub, sub)]],
                    gb.at[h, j], gsems[h])

        def gdrain(h, si):
            for j in range(6):
                pltpu.make_async_copy(
                    lat_hbm.at[idx_v.at[j, pl.ds(si * sub, sub)]],
                    gb.at[h, j], gsems[h]).wait()

        def compute(h):
            k0, p0, n0, k1, p1, n1 = (gb.at[h, j] for j in range(6))
            pbuf = ob.at[h, 0]
            nbuf = ob.at[h, 1]

            def row_body(r, inner):
                for j in range(half // LANES):
                    sl = pl.ds(j * LANES, LANES)
                    kv0 = k0[r, sl]
                    kv1 = k1[r, sl]
                    pbuf[r, sl] = kv0 * p0[r, sl] + kv1 * p1[r, sl]
                    nbuf[r, sl] = kv0 * n0[r, sl] + kv1 * n1[r, sl]
                return inner

            lax.fori_loop(0, sub, row_body, 0, unroll=4)

        def owait(h, si):
            pltpu.make_async_copy(
                ob.at[h, 0], psp_hbm.at[pl.ds(base + si * sub, sub)],
                psems[h]).wait()
            pltpu.make_async_copy(
                ob.at[h, 1], nsp_hbm.at[pl.ds(base + si * sub, sub)],
                nsems[h]).wait()

        def ofire(h, si):
            pltpu.async_copy(ob.at[h, 0],
                             psp_hbm.at[pl.ds(base + si * sub, sub)],
                             psems[h])
            pltpu.async_copy(ob.at[h, 1],
                             nsp_hbm.at[pl.ds(base + si * sub, sub)],
                             nsems[h])

        gfire(0, 0)

        def pair_body(q, carry):
            a = 2 * q
            gdrain(0, a)
            gfire(1, a + 1)

            @pl.when(q >= 1)
            def _w0():
                owait(0, a - 2)

            compute(0)
            ofire(0, a)
            gdrain(1, a + 1)

            @pl.when(q + 1 < nsub // 2)
            def _g0():
                gfire(0, a + 2)

            @pl.when(q >= 1)
            def _w1():
                owait(1, a - 1)

            compute(1)
            ofire(1, a + 1)
            return carry

        lax.fori_loop(0, nsub // 2, pair_body, 0)
        owait(0, nsub - 2)
        owait(1, nsub - 1)

    return k(lat_flat, idx6)


def _metrics(psp3, nsp3):
    m, r, c = psp3.shape

    def body(ps_ref, ns_ref, out_ref):
        ps = jnp.sum(ps_ref[...], axis=2)
        ns = jnp.sum(ns_ref[...], axis=2)
        d = ps - ns
        e = jnp.exp(-jnp.abs(d))
        sig = jnp.where(d >= 0, 1.0 / (1.0 + e), e / (1.0 + e))
        lterm = jnp.log(sig + 1e-9)
        nb = jnp.float32(m * r)
        loss = -jnp.sum(lterm) / nb
        win = ps >= ns
        hr = jnp.sum(win.astype(jnp.float32)) / nb
        mrr = jnp.sum(jnp.where(win, jnp.float32(1e-9),
                                jnp.float32(1.0))) / nb
        ndcg = jnp.sum(jnp.where(win, jnp.float32(1.0),
                                 jnp.float32(2.0 / 3.0))) / nb
        row = lax.broadcasted_iota(jnp.int32, (8, 128), 0)
        lane = lax.broadcasted_iota(jnp.int32, (8, 128), 1)
        z = jnp.zeros((8, 128), jnp.float32)
        v = jnp.where((row == 0) & (lane == 0), loss, z)
        v = jnp.where((row == 0) & (lane == 1), mrr, v)
        v = jnp.where((row == 0) & (lane == 2), hr, v)
        v = jnp.where((row == 0) & (lane == 3), ndcg, v)
        out_ref[...] = v

    return pl.pallas_call(
        body,
        in_specs=[pl.BlockSpec((m, r, c), lambda: (0, 0, 0))] * 2,
        out_specs=pl.BlockSpec((8, 128), lambda: (0, 0)),
        out_shape=jax.ShapeDtypeStruct((8, 128), jnp.float32),
    )(psp3, nsp3)


def kernel(features, edge_index, edge_weight, train_set, W_emb, b_emb,
           W_gc1, b_gc1):
    n, _ = features.shape
    d = W_emb.shape[1]
    half = d // 2
    e = edge_index.shape[1]
    b = train_set.shape[0]

    support3 = _dense_support(features, W_emb, b_emb, W_gc1)
    sflat = support3.reshape(2 * n, half)

    ck = 80
    blk = ck * 32 * NS
    e2 = (e + blk - 1) // blk * blk
    pad = e2 - e
    rows = jnp.concatenate([edge_index[0], jnp.zeros((pad,), jnp.int32)])
    cols = jnp.concatenate([edge_index[1], jnp.zeros((pad,), jnp.int32)])
    wpad = jnp.concatenate([edge_weight, jnp.zeros((pad,), jnp.float32)])
    ept = e2 // NS
    nch = ept // ck
    rows3 = rows.reshape(NS, nch, ck)
    cols4 = jnp.stack([cols, cols + n]).reshape(NC, NS, nch, ck)
    ztile = jnp.zeros(((n // NS) // 8 * 8, half), jnp.float32)

    w16 = jnp.broadcast_to(wpad[:, None], (e2, LANES))
    as_flat = _spmm_sc(rows3, cols4, w16, sflat, ztile, n)
    aas_flat = _spmm_sc(rows3, cols4, w16, as_flat, ztile, n)

    latent3 = _mix(support3, as_flat.reshape(2, n, half),
                   aas_flat.reshape(2, n, half), b_gc1.reshape(2, 1, half))

    kk = train_set[:, 0]
    pp = train_set[:, 1]
    nn = train_set[:, 2]
    idx6 = jnp.stack([kk, pp, nn, kk + n, pp + n, nn + n])
    psp, nsp = _bpr_sc(latent3.reshape(2 * n, half), idx6, b)

    out8 = _metrics(psp.reshape(b // 128, 128, half),
                    nsp.reshape(b // 128, 128, half))
    return (out8[0, 0], out8[0, 1], out8[0, 2], out8[0, 3])
